# baseline (speedup 1.0000x reference)
import functools

import jax
import jax.numpy as jnp
from jax import lax
from jax.experimental import pallas as pl
from jax.experimental.pallas import tpu as pltpu
from jax.experimental.pallas import tpu_sc as plsc

NC = 2
NS = 16
L = 16
NW = NC * NS

CH = 128


def _make_sc_kernel(B, K):
    assert B % NW == 0
    bw = B // NW
    nch = bw // CH
    assert nch * CH == bw and K % L == 0

    mesh = plsc.VectorSubcoreMesh(core_axis_name="c", subcore_axis_name="s")

    @functools.partial(
        pl.kernel,
        mesh=mesh,
        out_type=jax.ShapeDtypeStruct((B,), jnp.float32),
        compiler_params=pltpu.CompilerParams(
            needs_layout_passes=False, use_tc_tiling_on_sc=False),
        scratch_types=[
            pltpu.VMEM((nch, CH), jnp.int32),
            pltpu.VMEM((nch, CH), jnp.int32),
            pltpu.VMEM((CH, K), jnp.float32),
            pltpu.VMEM((CH, K), jnp.float32),
            pltpu.VMEM((bw,), jnp.float32),
            pltpu.SemaphoreType.DMA,
        ],
    )
    def body(users_hbm, movies_hbm, uemb_hbm, memb_hbm, out_hbm,
             uidx, midx, urows, mrows, outv, sem):
        wid = lax.axis_index("s") * NC + lax.axis_index("c")
        pltpu.sync_copy(users_hbm.at[wid], uidx)
        pltpu.sync_copy(movies_hbm.at[wid], midx)
        lane = lax.iota(jnp.int32, 16)

        for c in range(nch):
            pltpu.async_copy(uemb_hbm.at[uidx.at[c]], urows, sem).wait()
            pltpu.async_copy(memb_hbm.at[midx.at[c]], mrows, sem).wait()

            def group(g, _, c=c):
                accv = jnp.zeros((L,), jnp.float32)
                for i in range(L):
                    r = g * L + i
                    p = urows[r, pl.ds(0, L)] * mrows[r, pl.ds(0, L)]
                    for k in range(L, K, L):
                        p += urows[r, pl.ds(k, L)] * mrows[r, pl.ds(k, L)]
                    accv = jnp.where(lane == i, plsc.cumsum(p)[15], accv)
                outv[pl.ds(c * CH + g * L, L)] = accv
                return _

            lax.fori_loop(0, CH // L, group, 0)

        pltpu.sync_copy(outv, out_hbm.at[pl.ds(wid * bw, bw)])

    return body


def kernel(users, movies, user_emb, movie_emb):
    B = users.shape[0]
    K = user_emb.shape[1]
    bw = B // NW
    nch = bw // CH
    u3 = users.astype(jnp.int32).reshape(NW, nch, CH)
    m3 = movies.astype(jnp.int32).reshape(NW, nch, CH)
    return _make_sc_kernel(B, K)(u3, m3, user_emb, movie_emb)


# speedup vs baseline: 1.5898x; 1.5898x over previous
import functools

import jax
import jax.numpy as jnp
from jax import lax
from jax.experimental import pallas as pl
from jax.experimental.pallas import tpu as pltpu
from jax.experimental.pallas import tpu_sc as plsc

NC = 2
NS = 16
L = 16
NW = NC * NS


def _make_sc_kernel(B, K):
    assert B % NW == 0
    bw = B // NW
    ng = bw // L
    assert ng * L == bw and K % L == 0

    mesh = plsc.VectorSubcoreMesh(core_axis_name="c", subcore_axis_name="s")

    @functools.partial(
        pl.kernel,
        mesh=mesh,
        out_type=jax.ShapeDtypeStruct((B,), jnp.float32),
        compiler_params=pltpu.CompilerParams(
            needs_layout_passes=False, use_tc_tiling_on_sc=True),
        scratch_types=[
            pltpu.VMEM((bw,), jnp.int32),
            pltpu.VMEM((bw,), jnp.int32),
            pltpu.VMEM((L, K), jnp.float32),
            pltpu.VMEM((L, K), jnp.float32),
            pltpu.VMEM((bw,), jnp.float32),
            pltpu.SemaphoreType.DMA,
        ],
    )
    def body(users_hbm, movies_hbm, uemb_hbm, memb_hbm, out_hbm,
             uidx, midx, urows, mrows, outv, sem):
        wid = lax.axis_index("s") * NC + lax.axis_index("c")
        base = wid * bw
        pltpu.sync_copy(users_hbm.at[pl.ds(base, bw)], uidx)
        pltpu.sync_copy(movies_hbm.at[pl.ds(base, bw)], midx)
        lane = lax.iota(jnp.int32, L)

        def group(g, _):
            uvec = uidx[pl.ds(g * L, L)]
            mvec = midx[pl.ds(g * L, L)]
            copies = []
            for i in range(L):
                copies.append(
                    pltpu.async_copy(uemb_hbm.at[uvec[i]], urows.at[i], sem))
                copies.append(
                    pltpu.async_copy(memb_hbm.at[mvec[i]], mrows.at[i], sem))
            for c in copies:
                c.wait()
            accv = jnp.zeros((L,), jnp.float32)
            for i in range(L):
                p = urows[i, pl.ds(0, L)] * mrows[i, pl.ds(0, L)]
                for k in range(L, K, L):
                    p += urows[i, pl.ds(k, L)] * mrows[i, pl.ds(k, L)]
                accv = jnp.where(lane == i, plsc.cumsum(p)[L - 1], accv)
            outv[pl.ds(g * L, L)] = accv
            return _

        lax.fori_loop(0, ng, group, 0)
        pltpu.sync_copy(outv, out_hbm.at[pl.ds(base, bw)])

    return body


def kernel(users, movies, user_emb, movie_emb):
    B = users.shape[0]
    K = user_emb.shape[1]
    return _make_sc_kernel(B, K)(
        users.astype(jnp.int32), movies.astype(jnp.int32),
        user_emb, movie_emb)
